# baseline (speedup 1.0000x reference)
import functools

import jax
import jax.numpy as jnp
from jax import lax
from jax.experimental import pallas as pl
from jax.experimental.pallas import tpu as pltpu
from jax.experimental.pallas import tpu_sc as plsc

D = 64
CHUNK = 128
NC = 2
NS = 16
NW = NC * NS


def _make_gather(n_chunks_per_w: int, group: int):
    n_groups = n_chunks_per_w // group
    mesh = plsc.VectorSubcoreMesh(core_axis_name="c", subcore_axis_name="s")

    @functools.partial(
        pl.kernel,
        mesh=mesh,
        out_type=jax.ShapeDtypeStruct((NW, n_chunks_per_w, CHUNK, D), jnp.float32),
        scratch_types=[
            pltpu.VMEM((n_chunks_per_w, CHUNK), jnp.int32),
            pltpu.VMEM((group, CHUNK, D), jnp.float32),
            pltpu.SemaphoreType.DMA,
        ],
        compiler_params=pltpu.CompilerParams(use_tc_tiling_on_sc=False),
    )
    def gather(table_hbm, idx_hbm, out_hbm, idx_v, rows_v, sem):
        wid = lax.axis_index("s") * NC + lax.axis_index("c")
        pltpu.sync_copy(idx_hbm.at[wid], idx_v)

        def grp(g, carry):
            base = g * group
            cps = [
                pltpu.async_copy(
                    table_hbm.at[idx_v.at[base + j]], rows_v.at[j], sem
                )
                for j in range(group)
            ]
            for cp in cps:
                cp.wait()
            pltpu.sync_copy(rows_v, out_hbm.at[wid, pl.ds(base, group)])
            return carry

        lax.fori_loop(0, n_groups, grp, 0)

    return gather


@jax.jit
def kernel(input, table):
    b, s = input.shape
    n = b * s
    per_w = n // NW
    n_chunks = per_w // CHUNK
    idx = input.astype(jnp.int32).reshape(NW, n_chunks, CHUNK)
    out = _make_gather(n_chunks, 5)(table, idx)
    return out.reshape(b, s, D)


# speedup vs baseline: 1.0099x; 1.0099x over previous
import functools

import jax
import jax.numpy as jnp
from jax import lax
from jax.experimental import pallas as pl
from jax.experimental.pallas import tpu as pltpu
from jax.experimental.pallas import tpu_sc as plsc

D = 64
CHUNK = 128
NC = 2
NS = 16
NW = NC * NS
NBUF = 5
G = 2


def _make_gather(n_chunks_per_w: int):
    n_groups = n_chunks_per_w // G
    n_iters = n_groups // NBUF
    mesh = plsc.VectorSubcoreMesh(core_axis_name="c", subcore_axis_name="s")

    @functools.partial(
        pl.kernel,
        mesh=mesh,
        out_type=jax.ShapeDtypeStruct((NW, n_chunks_per_w, CHUNK, D), jnp.float32),
        scratch_types=[
            pltpu.VMEM((n_chunks_per_w, CHUNK), jnp.int32),
            pltpu.VMEM((NBUF, G, CHUNK, D), jnp.float32),
        ]
        + [pltpu.SemaphoreType.DMA] * (2 * NBUF),
        compiler_params=pltpu.CompilerParams(use_tc_tiling_on_sc=False),
    )
    def gather(table_hbm, idx_hbm, out_hbm, idx_v, rows_v, *sems):
        gsem = sems[:NBUF]
        wsem = sems[NBUF:]
        wid = lax.axis_index("s") * NC + lax.axis_index("c")
        pltpu.sync_copy(idx_hbm.at[wid], idx_v)

        def fire(g, b):
            for j in range(G):
                pltpu.async_copy(
                    table_hbm.at[idx_v.at[g * G + j]], rows_v.at[b, j], gsem[b]
                )

        def drain_gathers(g, b):
            for j in range(G):
                pltpu.make_async_copy(
                    table_hbm.at[idx_v.at[g * G + j]], rows_v.at[b, j], gsem[b]
                ).wait()

        def fire_write(g, b):
            pltpu.async_copy(
                rows_v.at[b], out_hbm.at[wid, pl.ds(g * G, G)], wsem[b]
            )

        def drain_write(g, b):
            pltpu.make_async_copy(
                rows_v.at[b], out_hbm.at[wid, pl.ds(g * G, G)], wsem[b]
            ).wait()

        for b in range(NBUF):
            fire(b, b)

        def body(it, carry):
            for b in range(NBUF):
                g = it * NBUF + b
                drain_gathers(g, b)
                fire_write(g, b)
                drain_write(g, b)
                fire(g + NBUF, b)
            return carry

        lax.fori_loop(0, n_iters - 1, body, 0)

        last = (n_iters - 1) * NBUF
        for b in range(NBUF):
            g = last + b
            drain_gathers(g, b)
            fire_write(g, b)
        for b in range(NBUF):
            drain_write(last + b, b)

    return gather


@jax.jit
def kernel(input, table):
    b, s = input.shape
    n = b * s
    per_w = n // NW
    n_chunks = per_w // CHUNK
    idx = input.astype(jnp.int32).reshape(NW, n_chunks, CHUNK)
    out = _make_gather(n_chunks)(table, idx)
    return out.reshape(b, s, D)
